# baseline (speedup 1.0000x reference)
import functools
import math

import jax
import jax.numpy as jnp
from jax import lax
from jax.experimental import pallas as pl
from jax.experimental.pallas import tpu as pltpu
from jax.experimental.pallas import tpu_sc as plsc

EMB = 128
N0 = 10000
E = 320000
B = 8
NPER0 = N0 // B
BN_EPS = 1e-5
NUM_LAYERS = 3

LAYER_N = [10000, 5000, 2504]
LAYER_NPER = [1250, 625, 313]
LAYER_K = [625, 313]


def _npad(n):
    return 1280 * ((n + 1 + 1279) // 1280)


NP_L = [_npad(n) for n in LAYER_N]
NEG_BIG = -3.0e38


def _bn(x, g, b):
    return g * x / jnp.sqrt(1.0 + BN_EPS) + b


def _mm_body(a_ref, b_ref, o_ref):
    o_ref[...] = jnp.dot(a_ref[...], b_ref[...],
                         preferred_element_type=jnp.float32)


def _mm(a, b):
    M, K = a.shape
    Nc = b.shape[1]
    Mp = ((M + 511) // 512) * 512
    Ncp = max(Nc, 128)
    if Mp != M:
        a = jnp.pad(a, ((0, Mp - M), (0, 0)))
    if Ncp != Nc:
        b = jnp.pad(b, ((0, 0), (0, Ncp - Nc)))
    out = pl.pallas_call(
        _mm_body,
        grid=(Mp // 512,),
        in_specs=[
            pl.BlockSpec((512, K), lambda i: (i, 0)),
            pl.BlockSpec((K, Ncp), lambda i: (0, 0)),
        ],
        out_specs=pl.BlockSpec((512, Ncp), lambda i: (i, 0)),
        out_shape=jax.ShapeDtypeStruct((Mp, Ncp), jnp.float32),
    )(a, b)
    return out[:M, :Nc]


def _split3(t):
    def trunc(v):
        return jax.lax.bitcast_convert_type(
            jax.lax.bitcast_convert_type(v, jnp.uint32) & jnp.uint32(0xFFFF0000),
            jnp.float32)
    p1 = trunc(t)
    r = t - p1
    p2 = trunc(r)
    p3 = r - p2
    return jnp.stack([p1, p2, p3])


def _exact_sel(oh, t1, t2, t3):
    d1 = jnp.dot(oh, t1, preferred_element_type=jnp.float32)
    d2 = jnp.dot(oh, t2, preferred_element_type=jnp.float32)
    d3 = jnp.dot(oh, t3, preferred_element_type=jnp.float32)
    return (d1 + d2) + d3


def _atom_body(x_ref, emb_ref, o_ref):
    o_ref[...] = jnp.zeros((512, EMB), jnp.float32)
    for f in range(9):
        xc = x_ref[:, f:f + 1]
        oh = (jax.lax.broadcasted_iota(jnp.int32, (512, 16), 1) == xc
              ).astype(jnp.float32)
        g = _exact_sel(oh, emb_ref[0, f], emb_ref[1, f], emb_ref[2, f])
        o_ref[...] = o_ref[...] + g


def _atom_encode(xp, atom_emb3):
    npr = xp.shape[0]
    return pl.pallas_call(
        _atom_body,
        grid=(npr // 512,),
        in_specs=[pl.BlockSpec((512, 16), lambda i: (i, 0)),
                  pl.BlockSpec((3, 9, 16, EMB), lambda i: (0, 0, 0, 0))],
        out_specs=pl.BlockSpec((512, EMB), lambda i: (i, 0)),
        out_shape=jax.ShapeDtypeStruct((npr, EMB), jnp.float32),
    )(xp, atom_emb3)


def _t3_body(be_ref, o_ref):
    r = jax.lax.broadcasted_iota(jnp.int32, (512, 1), 0)
    feats = [r >> 6, (r >> 3) & 7, r & 7]
    o_ref[...] = jnp.zeros((512, EMB), jnp.float32)
    for f in range(3):
        g = jnp.zeros((512, EMB), jnp.float32)
        for v in range(8):
            g = g + jnp.where(feats[f] == v, be_ref[f, v:v + 1, :], 0.0)
        o_ref[...] = o_ref[...] + g


def _t3_build(bond_emb):
    return pl.pallas_call(
        _t3_body,
        out_shape=jax.ShapeDtypeStruct((512, EMB), jnp.float32),
    )(bond_emb)


def _c3_body(ea_ref, o_ref):
    o_ref[...] = ea_ref[0] * 64 + ea_ref[1] * 8 + ea_ref[2]


def _c3_build(edge_attr):
    eb = E // 128
    ea3 = edge_attr.T.reshape(3, eb, 128)
    out = pl.pallas_call(
        _c3_body,
        out_shape=jax.ShapeDtypeStruct((eb, 128), jnp.int32),
    )(ea3)
    return out.reshape(E)


def _vmesh():
    return plsc.VectorSubcoreMesh(core_axis_name="c", subcore_axis_name="s")


def _sc_params():
    import dataclasses
    cp = pltpu.CompilerParams()
    if "needs_layout_passes" in pltpu.CompilerParams.__dataclass_fields__:
        cp = dataclasses.replace(cp, needs_layout_passes=False)
    return cp


_CH = 80
_EPT = E // 32


def _zero_spmem(zbuf, shared, row0, nrows):
    @pl.loop(0, nrows, step=80)
    def _(r):
        pltpu.sync_copy(zbuf, shared.at[pl.ds(row0 + r, 80)])


def _dump_spmem(shared, out_ref, cid, row0, nrows):
    pltpu.sync_copy(shared.at[pl.ds(row0, nrows)],
                    out_ref.at[cid, pl.ds(row0, nrows)])


def _iota16():
    return jax.lax.iota(jnp.int32, 16)


def _msg_compute(hbuf, t3buf, c3b):
    for g in range(_CH // 16):
        erow = _iota16() + (16 * g)
        c3v = c3b[pl.ds(16 * g, 16)]

        @pl.loop(0, 128)
        def _(f):
            fv = jnp.zeros((16,), jnp.int32) + f
            hv = plsc.load_gather(hbuf, [erow, fv])
            ev = plsc.load_gather(t3buf, [c3v, fv])
            m = jnp.maximum(hv + ev, 0.0)
            plsc.store_scatter(hbuf, [erow, fv], m)


def _sc_msg(hin, srcv, dstv, c3, T3, NPc, with_deg, scatter_agg):
    outs = []
    if scatter_agg:
        outs.append(jax.ShapeDtypeStruct((2, NPc, EMB), jnp.float32))
    else:
        outs.append(jax.ShapeDtypeStruct((E, EMB), jnp.float32))
    if with_deg:
        outs.append(jax.ShapeDtypeStruct((32, NPc), jnp.float32))
    scratch = [
        pltpu.VMEM((512, EMB), jnp.float32),
        pltpu.VMEM((_CH, EMB), jnp.float32),
        pltpu.VMEM((_CH,), jnp.int32),
        pltpu.VMEM((_CH,), jnp.int32),
        pltpu.VMEM((_CH,), jnp.int32),
        pltpu.VMEM((NPc,), jnp.float32),
        pltpu.SemaphoreType.DMA,
    ]
    if scatter_agg:
        scratch.append(pltpu.VMEM_SHARED((NPc, EMB), jnp.float32))

    @functools.partial(
        pl.kernel, out_type=tuple(outs) if len(outs) > 1 else outs[0],
        mesh=_vmesh(), scratch_types=scratch,
        compiler_params=_sc_params())
    def k(hin_h, src_h, dst_h, c3_h, t3_h, *refs):
        refs = list(refs)
        if scatter_agg:
            agg_o = refs.pop(0)
        else:
            msg_o = refs.pop(0)
        deg_o = refs.pop(0) if with_deg else None
        t3buf, hbuf, sidx, didx, c3b, degT, sem = refs[:7]
        aggS = refs[7] if scatter_agg else None

        cid = lax.axis_index("c")
        sid = lax.axis_index("s")
        wid = cid * 16 + sid
        rows_per = NPc // 16
        row0 = sid * rows_per

        zv = jnp.zeros((16,), jnp.float32)

        @pl.loop(0, NPc, step=16)
        def _(r):
            degT[pl.ds(r, 16)] = zv

        if scatter_agg:
            @pl.loop(0, _CH)
            def _(r):
                for j in range(EMB // 16):
                    hbuf[r, pl.ds(16 * j, 16)] = zv
            _zero_spmem(hbuf, aggS, row0, rows_per)
            plsc.subcore_barrier()

        pltpu.sync_copy(t3_h, t3buf)
        onev = jnp.zeros((16,), jnp.float32) + 1.0

        @pl.loop(0, _EPT // _CH)
        def _(i):
            base = wid * _EPT + i * _CH
            pltpu.sync_copy(src_h.at[pl.ds(base, _CH)], sidx)
            pltpu.sync_copy(dst_h.at[pl.ds(base, _CH)], didx)
            pltpu.sync_copy(c3_h.at[pl.ds(base, _CH)], c3b)
            pltpu.async_copy(hin_h.at[sidx], hbuf, sem).wait()
            _msg_compute(hbuf, t3buf, c3b)
            if scatter_agg:
                pltpu.sync_copy(hbuf, aggS.at[didx], add=True)
            else:
                pltpu.sync_copy(hbuf, msg_o.at[pl.ds(base, _CH)])
            if with_deg:
                for g in range(_CH // 16):
                    dv = didx[pl.ds(16 * g, 16)]
                    plsc.addupdate_scatter(degT, [dv], onev)

        if with_deg:
            pltpu.sync_copy(degT, deg_o.at[wid])
        if scatter_agg:
            plsc.subcore_barrier()
            _dump_spmem(aggS, agg_o, cid, row0, rows_per)

    return k(hin, srcv, dstv, c3, T3)


def _sc_atom(x16, atom_emb):
    NPc = x16.shape[0]
    npt = NPc // 32
    nch = npt // _CH

    @functools.partial(
        pl.kernel, out_type=jax.ShapeDtypeStruct((NPc, EMB), jnp.float32),
        mesh=_vmesh(), compiler_params=_sc_params(),
        scratch_types=[
            pltpu.VMEM((9, 16, EMB), jnp.float32),
            pltpu.VMEM((_CH, 16), jnp.int32),
            pltpu.VMEM((_CH, EMB), jnp.float32),
        ])
    def k(x_h, emb_h, o_h, tab, xbuf, obuf):
        cid = lax.axis_index("c")
        sid = lax.axis_index("s")
        wid = cid * 16 + sid
        pltpu.sync_copy(emb_h, tab)

        @pl.loop(0, nch)
        def _(i):
            base = wid * npt + i * _CH
            pltpu.sync_copy(x_h.at[pl.ds(base, _CH)], xbuf)
            for g in range(_CH // 16):
                nrow = _iota16() + 16 * g
                xvs = [plsc.load_gather(xbuf,
                                        [nrow, jnp.full((16,), f, jnp.int32)])
                       for f in range(9)]

                @pl.loop(0, EMB)
                def _(c):
                    cv = jnp.zeros((16,), jnp.int32) + c
                    acc = jnp.zeros((16,), jnp.float32)
                    for f in range(9):
                        fv = jnp.full((16,), f, jnp.int32)
                        acc = acc + plsc.load_gather(tab, [fv, xvs[f], cv])
                    plsc.store_scatter(obuf, [nrow, cv], acc)
            pltpu.sync_copy(obuf, o_h.at[pl.ds(base, _CH)])

    return k(x16, atom_emb)


def _sc_score(u, srcv, dstv, NPc):
    @functools.partial(
        pl.kernel, out_type=jax.ShapeDtypeStruct((32, NPc), jnp.float32),
        mesh=_vmesh(), compiler_params=_sc_params(),
        scratch_types=[
            pltpu.VMEM((NPc,), jnp.float32),
            pltpu.VMEM((NPc,), jnp.float32),
            pltpu.VMEM((_CH,), jnp.int32),
            pltpu.VMEM((_CH,), jnp.int32),
        ])
    def k(u_h, src_h, dst_h, scp_o, ubuf, scT, sidx, didx):
        cid = lax.axis_index("c")
        sid = lax.axis_index("s")
        wid = cid * 16 + sid
        zv = jnp.zeros((16,), jnp.float32)

        @pl.loop(0, NPc, step=16)
        def _(r):
            scT[pl.ds(r, 16)] = zv

        pltpu.sync_copy(u_h, ubuf)

        @pl.loop(0, _EPT // _CH)
        def _(i):
            base = wid * _EPT + i * _CH
            pltpu.sync_copy(src_h.at[pl.ds(base, _CH)], sidx)
            pltpu.sync_copy(dst_h.at[pl.ds(base, _CH)], didx)
            for g in range(_CH // 16):
                sv = sidx[pl.ds(16 * g, 16)]
                dv = didx[pl.ds(16 * g, 16)]
                uv = plsc.load_gather(ubuf, [sv])
                plsc.addupdate_scatter(scT, [dv], uv)

        pltpu.sync_copy(scT, scp_o.at[wid])

    return k(u, srcv, dstv)


def _pre_body(nper, n, with_sg, with_vt, hg_ref, *refs):
    if with_sg:
        sg_ref = refs[0]
        refs = refs[1:]
    vn_ref = refs[0]
    hin_ref = refs[1]
    vt_ref = refs[2] if with_vt else None
    i = pl.program_id(0)
    x = hg_ref[...]
    if with_sg:
        x = x * sg_ref[...]
    rows = i * 512 + jax.lax.broadcasted_iota(jnp.int32, (512, 1), 0)
    vnsel = jnp.zeros((512, EMB), jnp.float32)
    masks = []
    for g in range(B):
        m = (rows >= g * nper) & (rows < (g + 1) * nper)
        masks.append(m)
        vnsel = vnsel + jnp.where(m, vn_ref[g:g + 1, :], 0.0)
    h_in = x + vnsel
    hin_ref[...] = h_in
    if with_vt:
        @pl.when(i == 0)
        def _():
            vt_ref[...] = jnp.zeros((B, EMB), jnp.float32)
        for g in range(B):
            s = jnp.sum(jnp.where(masks[g], h_in, 0.0), axis=0, keepdims=True)
            vt_ref[g:g + 1, :] = vt_ref[g:g + 1, :] + s


def _pre(hg, sg, vnB, nper, with_vt):
    npr = hg.shape[0]
    with_sg = sg is not None
    ins = [hg] + ([sg] if with_sg else []) + [vnB]
    in_specs = [pl.BlockSpec((512, EMB), lambda i: (i, 0))]
    if with_sg:
        in_specs.append(pl.BlockSpec((512, EMB), lambda i: (i, 0)))
    in_specs.append(pl.BlockSpec((B, EMB), lambda i: (0, 0)))
    out_shapes = [jax.ShapeDtypeStruct((npr, EMB), jnp.float32)]
    out_specs = [pl.BlockSpec((512, EMB), lambda i: (i, 0))]
    if with_vt:
        out_shapes.append(jax.ShapeDtypeStruct((B, EMB), jnp.float32))
        out_specs.append(pl.BlockSpec((B, EMB), lambda i: (0, 0)))
    res = pl.pallas_call(
        functools.partial(_pre_body, nper, npr, with_sg, with_vt),
        grid=(npr // 512,),
        in_specs=in_specs,
        out_specs=out_specs,
        out_shape=out_shapes,
    )(*ins)
    return (res[0], res[1]) if with_vt else (res[0], None)


def _mlp_body(with_score, hin_ref, aggp_ref, degp_ref, epsb_ref,
              w1_ref, b1_ref, g1_ref, bb1_ref,
              w2_ref, b2_ref, g2_ref, bb2_ref, sagw_ref,
              h_ref, xw_ref, dinv_ref, u_ref, last_relu):
    agg = aggp_ref[0] + aggp_ref[1]
    z = epsb_ref[...] * hin_ref[...] + agg
    z = _bn(jnp.dot(z, w1_ref[...], preferred_element_type=jnp.float32)
            + b1_ref[...], g1_ref[...], bb1_ref[...])
    z = jnp.maximum(z, 0.0)
    h = jnp.dot(z, w2_ref[...], preferred_element_type=jnp.float32) + b2_ref[...]
    h = _bn(h, g2_ref[...], bb2_ref[...])
    if last_relu:
        h = jnp.maximum(h, 0.0)
    h_ref[...] = h
    if with_score:
        xwf = jnp.dot(h, sagw_ref[...], preferred_element_type=jnp.float32)
        xwc = xwf[:, 0:1]
        xw_ref[...] = jnp.broadcast_to(xwc, (512, 16))
        deg = jnp.sum(degp_ref[...], axis=1, keepdims=True) + 1.0
        dinv = jnp.exp(-0.5 * jnp.log(deg))
        dinv_ref[...] = jnp.broadcast_to(dinv, (512, 16))
        u_ref[...] = jnp.broadcast_to(dinv * xwc, (512, 16))
    else:
        z16 = jnp.zeros((512, 16), jnp.float32)
        xw_ref[...] = z16
        dinv_ref[...] = z16
        u_ref[...] = z16


def _mlp(hin, aggp, degp, epsb, w1, b1, g1, bb1, w2, b2, g2, bb2, sagwp,
         with_score, last_relu):
    npr = hin.shape[0]
    in_specs = [
        pl.BlockSpec((512, EMB), lambda i: (i, 0)),
        pl.BlockSpec((2, 512, EMB), lambda i: (0, i, 0)),
        pl.BlockSpec((512, 32), lambda i: (i, 0)),
        pl.BlockSpec((1, EMB), lambda i: (0, 0)),
        pl.BlockSpec((EMB, EMB), lambda i: (0, 0)),
        pl.BlockSpec((1, EMB), lambda i: (0, 0)),
        pl.BlockSpec((1, EMB), lambda i: (0, 0)),
        pl.BlockSpec((1, EMB), lambda i: (0, 0)),
        pl.BlockSpec((EMB, EMB), lambda i: (0, 0)),
        pl.BlockSpec((1, EMB), lambda i: (0, 0)),
        pl.BlockSpec((1, EMB), lambda i: (0, 0)),
        pl.BlockSpec((1, EMB), lambda i: (0, 0)),
        pl.BlockSpec((EMB, EMB), lambda i: (0, 0)),
    ]
    out_shapes = [jax.ShapeDtypeStruct((npr, EMB), jnp.float32),
                  jax.ShapeDtypeStruct((npr, 16), jnp.float32),
                  jax.ShapeDtypeStruct((npr, 16), jnp.float32),
                  jax.ShapeDtypeStruct((npr, 16), jnp.float32)]
    out_specs = [pl.BlockSpec((512, EMB), lambda i: (i, 0)),
                 pl.BlockSpec((512, 16), lambda i: (i, 0)),
                 pl.BlockSpec((512, 16), lambda i: (i, 0)),
                 pl.BlockSpec((512, 16), lambda i: (i, 0))]

    def body(hin_ref, aggp_ref, degp_ref, epsb_ref, w1_ref, b1_ref, g1_ref,
             bb1_ref, w2_ref, b2_ref, g2_ref, bb2_ref, sagw_ref,
             h_ref, xw_ref, dinv_ref, u_ref):
        _mlp_body(with_score, hin_ref, aggp_ref, degp_ref, epsb_ref,
                  w1_ref, b1_ref, g1_ref, bb1_ref, w2_ref, b2_ref, g2_ref,
                  bb2_ref, sagw_ref, h_ref, xw_ref, dinv_ref, u_ref,
                  last_relu)

    return pl.pallas_call(
        body,
        grid=(npr // 512,),
        in_specs=in_specs,
        out_specs=out_specs,
        out_shape=out_shapes,
    )(hin, aggp, degp, epsb, w1, b1, g1, bb1, w2, b2, g2, bb2, sagwp)


def _vn_body(vts_ref, vnp_ref, w1_ref, b1_ref, g1_ref, bb1_ref,
             w2_ref, b2_ref, g2_ref, bb2_ref, o_ref):
    vt = vts_ref[...] + vnp_ref[...]
    t = _bn(jnp.dot(vt, w1_ref[...], preferred_element_type=jnp.float32)
            + b1_ref[...], g1_ref[...], bb1_ref[...])
    t = jnp.maximum(t, 0.0)
    t = _bn(jnp.dot(t, w2_ref[...], preferred_element_type=jnp.float32)
            + b2_ref[...], g2_ref[...], bb2_ref[...])
    o_ref[...] = jnp.maximum(t, 0.0)


def _vn_mlp(vtsum, vn_prev, w1, b1, g1, bb1, w2, b2, g2, bb2):
    return pl.pallas_call(
        _vn_body,
        out_shape=jax.ShapeDtypeStruct((B, EMB), jnp.float32),
    )(vtsum, vn_prev, w1.reshape(EMB, EMB), b1.reshape(1, EMB),
      g1.reshape(1, EMB), bb1.reshape(1, EMB), w2.reshape(EMB, EMB),
      b2.reshape(1, EMB), g2.reshape(1, EMB), bb2.reshape(1, EMB))


def _scorefin_body(scp_ref, dinv_ref, xw_ref, sagb_ref, sc_ref, sco_ref):
    sca = jnp.sum(scp_ref[...], axis=1, keepdims=True)
    dinv = dinv_ref[...]
    sc = dinv * sca + (dinv * dinv) * xw_ref[...] + sagb_ref[...]
    sc_ref[...] = sc
    sco_ref[...] = jnp.tanh(sc)


def _scorefin(scp, dinv16, xw16, sagb):
    npr = dinv16.shape[0]
    return pl.pallas_call(
        _scorefin_body,
        grid=(npr // 512,),
        in_specs=[pl.BlockSpec((512, 32), lambda i: (i, 0)),
                  pl.BlockSpec((512, 16), lambda i: (i, 0)),
                  pl.BlockSpec((512, 16), lambda i: (i, 0)),
                  pl.BlockSpec((1, 16), lambda i: (0, 0))],
        out_specs=[pl.BlockSpec((512, 16), lambda i: (i, 0)),
                   pl.BlockSpec((512, 16), lambda i: (i, 0))],
        out_shape=[jax.ShapeDtypeStruct((npr, 16), jnp.float32),
                   jax.ShapeDtypeStruct((npr, 16), jnp.float32)],
    )(scp, dinv16, xw16, sagb)


def _rank_body(npad, k, sbg3_ref, st3_ref, o_ref):
    g = pl.program_id(0)
    ib = pl.program_id(1)
    icc = st3_ref[...]
    ohg = (jax.lax.broadcasted_iota(jnp.int32, (8, 8), 1) == g)
    for q in range(4):
        ic = jnp.sum(jnp.where(ohg, icc[q], 0.0), axis=1, keepdims=True)
        iidx = (ib * 32 + q * 8
                + jax.lax.broadcasted_iota(jnp.int32, (8, 1), 0))
        cnt = jnp.zeros((8, 128), jnp.int32)
        for jr in range(npad // 128):
            sl = sbg3_ref[:, jr, :]
            jidx = jr * 128 + jax.lax.broadcasted_iota(jnp.int32, (1, 128), 1)
            gt = sl > ic
            tie = (sl == ic) & (jidx < iidx)
            cnt = cnt + (gt | tie).astype(jnp.int32)
        rank = jnp.sum(cnt, axis=1, keepdims=True)
        inv = jnp.where(rank < k, g * k + rank, -1)
        o_ref[0, 0, q] = jnp.broadcast_to(inv, (8, 8))


def _rank(s_bg, npad, nper, k):
    sbg3 = s_bg.reshape(B, npad // 128, 128)
    st3 = s_bg.T.reshape(npad // 8, 8, B)
    out4 = pl.pallas_call(
        functools.partial(_rank_body, npad, k),
        grid=(B, npad // 32),
        in_specs=[pl.BlockSpec((1, npad // 128, 128), lambda g, ib: (g, 0, 0)),
                  pl.BlockSpec((4, 8, 8), lambda g, ib: (ib, 0, 0))],
        out_specs=pl.BlockSpec((1, 1, 4, 8, 8), lambda g, ib: (g, ib, 0, 0, 0)),
        out_shape=jax.ShapeDtypeStruct((B, npad // 32, 4, 8, 8), jnp.int32),
    )(sbg3, st3)
    inv_bg = out4[:, :, :, :, 0].reshape(B, npad)[:, :nper]
    return inv_bg.reshape(B * nper)


def kernel(x, edge_index, edge_attr, batch, atom_emb, bond_emb, vn0,
           gin_W1, gin_b1, gin_bn1_g, gin_bn1_b, gin_W2, gin_b2, gin_eps,
           bn_g, bn_b, vn_W1, vn_b1, vn_bn1_g, vn_bn1_b, vn_W2, vn_b2,
           vn_bn2_g, vn_bn2_b, sag_W, sag_b):
    NP0 = NP_L[0]
    x16 = jnp.pad(x.astype(jnp.int32), ((0, NP0 - N0), (0, 16 - 9)))
    h0p = _sc_atom(x16, atom_emb)
    T3 = _t3_build(bond_emb)
    c3 = _c3_build(edge_attr.astype(jnp.int32))
    sagwp = jnp.pad(sag_W, ((0, 0), (0, EMB - 1)))
    sagb16 = jnp.broadcast_to(sag_b.reshape(1, 1), (1, 16))

    vn = jnp.broadcast_to(vn0, (B, EMB))
    src = edge_index[0].astype(jnp.int32)
    dst = edge_index[1].astype(jnp.int32)

    hg = h0p
    sgb = None
    h_list = []
    b_list = [jnp.repeat(jnp.arange(B, dtype=jnp.int32), NPER0)]
    h_last = None

    for layer in range(NUM_LAYERS):
        N = LAYER_N[layer]
        NPc = NP_L[layer]
        nper = LAYER_NPER[layer]
        with_score = layer < NUM_LAYERS - 1

        bcur = jnp.repeat(jnp.arange(B, dtype=jnp.int32), nper)
        vnfull = jnp.zeros((NPc, EMB), jnp.float32).at[:N].set(vn[bcur])
        if sgb is None:
            h_in = hg + vnfull
        else:
            h_in = hg * sgb + vnfull
        if with_score:
            vtsum = jax.ops.segment_sum(h_in[:N], bcur, num_segments=B)
        h_list.append(h_in[:N])

        if with_score:
            msg, degp32 = _sc_msg(h_in, src, dst, c3, T3, NPc,
                                  with_deg=True, scatter_agg=False)
            agg = jax.ops.segment_sum(msg, dst, num_segments=NPc)
            aggp = jnp.stack([agg, jnp.zeros_like(agg)])
            degt = degp32.T
        else:
            aggp = _sc_msg(h_in, src, dst, c3, T3, NPc,
                           with_deg=False, scatter_agg=True)
            degt = jnp.zeros((NPc, 32), jnp.float32)

        epsb = jnp.broadcast_to((1.0 + gin_eps[layer]).reshape(1, 1),
                                (1, EMB))
        if with_score:
            agg_c = aggp[0] + aggp[1]
            z = (1.0 + gin_eps[layer]) * h_in + agg_c
            z = _bn(_mm(z, gin_W1[layer]) + gin_b1[layer], gin_bn1_g[layer],
                    gin_bn1_b[layer])
            z = jax.nn.relu(z)
            h = _mm(z, gin_W2[layer]) + gin_b2[layer]
            h = jax.nn.relu(_bn(h, bn_g[layer], bn_b[layer]))
            xwc = _mm(h, sag_W)
            deg_c = jnp.sum(degt, axis=1, keepdims=True) + 1.0
            dinv_c = deg_c ** -0.5
            xw16 = jnp.broadcast_to(xwc, (NPc, 16))
            dinv16 = jnp.broadcast_to(dinv_c, (NPc, 16))
            u16 = jnp.broadcast_to(dinv_c * xwc, (NPc, 16))
        else:
            h, xw16, dinv16, u16 = _mlp(
                h_in, aggp, degt, epsb,
                gin_W1[layer], gin_b1[layer].reshape(1, EMB),
                gin_bn1_g[layer].reshape(1, EMB),
                gin_bn1_b[layer].reshape(1, EMB),
                gin_W2[layer], gin_b2[layer].reshape(1, EMB),
                bn_g[layer].reshape(1, EMB), bn_b[layer].reshape(1, EMB),
                sagwp, with_score=with_score, last_relu=with_score)

        if not with_score:
            h_last = h[:N]
            b_list.append(b_list[-1])
            break

        vt = vtsum + vn
        t = _bn(_mm(vt, vn_W1[layer]) + vn_b1[layer], vn_bn1_g[layer],
                vn_bn1_b[layer])
        t = jax.nn.relu(t)
        t = _bn(_mm(t, vn_W2[layer]) + vn_b2[layer], vn_bn2_g[layer],
                vn_bn2_b[layer])
        vn = jax.nn.relu(t)

        u = u16[:, 0]
        scp = _sc_score(u, src, dst, NPc).T

        sca = jnp.sum(scp, axis=1, keepdims=True)
        sc_c = (dinv16[:, 0:1] * sca + (dinv16[:, 0:1] ** 2) * xw16[:, 0:1]
                + sag_b)
        score_col = jnp.tanh(sc_c[:, 0])

        k = LAYER_K[layer]
        Nnew = B * k
        NPnew = NP_L[layer + 1]
        npad = 128 * ((nper + 127) // 128)
        s_bg = jnp.full((B, npad), NEG_BIG, jnp.float32
                        ).at[:, :nper].set(score_col[:N].reshape(B, nper))
        inv_n = _rank(s_bg, npad, nper, k)
        inv = jnp.pad(inv_n, (0, NPc - N), constant_values=-1)

        perm_t = jnp.where(inv_n >= 0, inv_n, Nnew)
        perm = jnp.zeros((Nnew + 1,), jnp.int32).at[perm_t].set(
            jnp.arange(N, dtype=jnp.int32))[:Nnew]
        permp = jnp.pad(perm, (0, NPnew - Nnew))
        hg = h[permp]
        score_n = score_col[:N]
        sgb = score_n[permp][:, None]
        vr = inv[src]
        vc = inv[dst]
        src = jnp.where(vr >= 0, vr, 0)
        dst = jnp.where((vr >= 0) & (vc >= 0), vc, Nnew)
        b_list.append(jnp.repeat(jnp.arange(B, dtype=jnp.int32), k))

    return tuple(h_list) + (h_last,) + tuple(b_list)
